# initial kernel scaffold (unmeasured)
import jax
import jax.numpy as jnp
from jax import lax
from jax.experimental import pallas as pl
from jax.experimental.pallas import tpu as pltpu

N_Z = 4
H, Dh, Dr = 16, 128, 32
DC = 128
SCALE = (Dh + Dr) ** -0.5
F32 = jnp.float32


def _matmul(a, b):
    m, _ = a.shape
    _, n = b.shape

    def body(a_ref, b_ref, o_ref):
        o_ref[...] = jnp.dot(a_ref[...], b_ref[...], preferred_element_type=F32)

    return pl.pallas_call(
        body, out_shape=jax.ShapeDtypeStruct((m, n), F32)
    )(a, b)


def _proj_qk(x2, Wq, Wqr, Wkr):
    m = x2.shape[0]

    def body(x_ref, wq_ref, wqr_ref, wkr_ref, q_ref, qr_ref, kr_ref):
        xv = x_ref[...]
        q_ref[...] = jnp.dot(xv, wq_ref[...], preferred_element_type=F32)
        qr_ref[...] = jnp.dot(xv, wqr_ref[...], preferred_element_type=F32)
        kr_ref[...] = jnp.dot(xv, wkr_ref[...], preferred_element_type=F32)

    return pl.pallas_call(
        body,
        out_shape=(
            jax.ShapeDtypeStruct((m, H * Dh), F32),
            jax.ShapeDtypeStruct((m, H * Dr), F32),
            jax.ShapeDtypeStruct((m, Dr), F32),
        ),
    )(x2, Wq, Wqr, Wkr)


def _gather_z(c, Wuk, Wuv):
    s, dc = c.shape
    _, d = Wuk.shape

    def body(c_ref, wuk_ref, wuv_ref, cg_ref, wukg_ref, wuvg_ref,
             send_sems, recv_sems):
        mx = lax.axis_index("x")
        my = lax.axis_index("y")
        mz = lax.axis_index("z")
        left = (mz + N_Z - 1) % N_Z
        right = (mz + 1) % N_Z

        barrier = pltpu.get_barrier_semaphore()
        for nbr in (left, right):
            pl.semaphore_signal(
                barrier, inc=1,
                device_id=(mx, my, nbr),
                device_id_type=pl.DeviceIdType.MESH,
            )
        pl.semaphore_wait(barrier, 2)

        cg_ref[:, pl.ds(mz * dc, dc)] = c_ref[...]
        wukg_ref[pl.ds(mz * dc, dc), :] = wuk_ref[...]
        wuvg_ref[pl.ds(mz * dc, dc), :] = wuv_ref[...]

        for h in range(N_Z - 1):
            origin = (mz - h) % N_Z
            rdmas = []
            for t, ref in enumerate((cg_ref, wukg_ref, wuvg_ref)):
                if t == 0:
                    sl = ref.at[:, pl.ds(origin * dc, dc)]
                else:
                    sl = ref.at[pl.ds(origin * dc, dc), :]
                rdma = pltpu.make_async_remote_copy(
                    src_ref=sl,
                    dst_ref=sl,
                    send_sem=send_sems.at[h, t],
                    recv_sem=recv_sems.at[h, t],
                    device_id=(mx, my, right),
                    device_id_type=pl.DeviceIdType.MESH,
                )
                rdma.start()
                rdmas.append(rdma)
            for rdma in rdmas:
                rdma.wait()

    return pl.pallas_call(
        body,
        out_shape=(
            jax.ShapeDtypeStruct((s, N_Z * dc), F32),
            jax.ShapeDtypeStruct((N_Z * dc, d), F32),
            jax.ShapeDtypeStruct((N_Z * dc, d), F32),
        ),
        scratch_shapes=[
            pltpu.SemaphoreType.DMA((N_Z - 1, 3)),
            pltpu.SemaphoreType.DMA((N_Z - 1, 3)),
        ],
        compiler_params=pltpu.CompilerParams(collective_id=0),
    )(c, Wuk, Wuv)


def _attention(q, k, v, qr, kr):
    s = q.shape[0]

    def body(q_ref, k_ref, v_ref, qr_ref, kr_ref, o_ref):
        sc = lax.dot_general(
            q_ref[...], k_ref[...], (((1,), (1,)), ((), ())),
            preferred_element_type=F32,
        )
        sc += lax.dot_general(
            qr_ref[...], kr_ref[...], (((1,), (1,)), ((), ())),
            preferred_element_type=F32,
        )
        sc *= SCALE
        m = jnp.max(sc, axis=1, keepdims=True)
        p = jnp.exp(sc - m)
        p = p / jnp.sum(p, axis=1, keepdims=True)
        o_ref[...] = jnp.dot(p, v_ref[...], preferred_element_type=F32)

    return pl.pallas_call(
        body,
        grid=(H,),
        in_specs=[
            pl.BlockSpec((s, Dh), lambda h: (0, h)),
            pl.BlockSpec((s, Dh), lambda h: (0, h)),
            pl.BlockSpec((s, Dh), lambda h: (0, h)),
            pl.BlockSpec((s, Dr), lambda h: (0, h)),
            pl.BlockSpec((s, Dr), lambda h: (0, 0)),
        ],
        out_specs=pl.BlockSpec((s, Dh), lambda h: (0, h)),
        out_shape=jax.ShapeDtypeStruct((s, H * Dh), F32),
    )(q, k, v, qr, kr)


def kernel(x, Wdkv, Wuk, Wuv, Wq, Wqr, Wkr, Wo):
    b, s, d = x.shape
    x2 = x.reshape(s, d)

    c = _matmul(x2, Wdkv)
    cg, wukg, wuvg = _gather_z(c, Wuk, Wuv)
    q, qr, kr = _proj_qk(x2, Wq, Wqr, Wkr)
    k = _matmul(cg, wukg)
    v = _matmul(cg, wuvg)
    o = _attention(q, k, v, qr, kr)
    out = _matmul(o, Wo)
    return out.reshape(b, s, d)


# baseline (device time: 203087 ns/iter reference)
import jax
import jax.numpy as jnp
from jax import lax
from jax.experimental import pallas as pl
from jax.experimental.pallas import tpu as pltpu

N_Z = 4
H, Dh, Dr = 16, 128, 32
DC = 128
SCALE = (Dh + Dr) ** -0.5
F32 = jnp.float32


def _matmul(a, b):
    m, _ = a.shape
    _, n = b.shape

    def body(a_ref, b_ref, o_ref):
        o_ref[...] = jnp.dot(a_ref[...], b_ref[...], preferred_element_type=F32)

    return pl.pallas_call(
        body, out_shape=jax.ShapeDtypeStruct((m, n), F32)
    )(a, b)


def _proj_qk(x2, Wq, Wqr, Wkr):
    m = x2.shape[0]

    def body(x_ref, wq_ref, wqr_ref, wkr_ref, q_ref, qr_ref, kr_ref):
        xv = x_ref[...]
        q_ref[...] = jnp.dot(xv, wq_ref[...], preferred_element_type=F32)
        qr_ref[...] = jnp.dot(xv, wqr_ref[...], preferred_element_type=F32)
        kr_ref[...] = jnp.dot(xv, wkr_ref[...], preferred_element_type=F32)

    return pl.pallas_call(
        body,
        out_shape=(
            jax.ShapeDtypeStruct((m, H * Dh), F32),
            jax.ShapeDtypeStruct((m, H * Dr), F32),
            jax.ShapeDtypeStruct((m, Dr), F32),
        ),
    )(x2, Wq, Wqr, Wkr)


def _gather_z(c, Wuk, Wuv):
    s, dc = c.shape
    _, d = Wuk.shape

    def body(c_ref, wuk_ref, wuv_ref, cg_ref, wukg_ref, wuvg_ref,
             send_sems, recv_sems):
        mx = lax.axis_index("x")
        my = lax.axis_index("y")
        mz = lax.axis_index("z")
        left = (mz + N_Z - 1) % N_Z
        right = (mz + 1) % N_Z

        barrier = pltpu.get_barrier_semaphore()
        for nbr in (left, right):
            pl.semaphore_signal(
                barrier, inc=1,
                device_id=(mx, my, nbr),
                device_id_type=pl.DeviceIdType.MESH,
            )
        pl.semaphore_wait(barrier, 2)

        cg_ref[:, pl.ds(mz * dc, dc)] = c_ref[...]
        wukg_ref[pl.ds(mz * dc, dc), :] = wuk_ref[...]
        wuvg_ref[pl.ds(mz * dc, dc), :] = wuv_ref[...]

        for h in range(N_Z - 1):
            origin = (mz - h) % N_Z
            rdmas = []
            for t, ref in enumerate((cg_ref, wukg_ref, wuvg_ref)):
                if t == 0:
                    sl = ref.at[:, pl.ds(origin * dc, dc)]
                else:
                    sl = ref.at[pl.ds(origin * dc, dc), :]
                rdma = pltpu.make_async_remote_copy(
                    src_ref=sl,
                    dst_ref=sl,
                    send_sem=send_sems.at[h, t],
                    recv_sem=recv_sems.at[h, t],
                    device_id=(mx, my, right),
                    device_id_type=pl.DeviceIdType.MESH,
                )
                rdma.start()
                rdmas.append(rdma)
            for rdma in rdmas:
                rdma.wait()

    return pl.pallas_call(
        body,
        out_shape=(
            jax.ShapeDtypeStruct((s, N_Z * dc), F32),
            jax.ShapeDtypeStruct((N_Z * dc, d), F32),
            jax.ShapeDtypeStruct((N_Z * dc, d), F32),
        ),
        scratch_shapes=[
            pltpu.SemaphoreType.DMA((N_Z - 1, 3)),
            pltpu.SemaphoreType.DMA((N_Z - 1, 3)),
        ],
        compiler_params=pltpu.CompilerParams(collective_id=0),
    )(c, Wuk, Wuv)


HB = 4


def _attention(q, k, v, qr, kr):
    s = q.shape[0]

    def body(q_ref, k_ref, v_ref, qr_ref, kr_ref, o_ref):
        kr_v = kr_ref[...]
        qr_v = qr_ref[...]
        for j in range(HB):
            q_h = q_ref[:, j * Dh:(j + 1) * Dh]
            k_h = k_ref[:, j * Dh:(j + 1) * Dh]
            qr_h = lax.slice(qr_v, (0, j * Dr), (s, (j + 1) * Dr))
            sc = lax.dot_general(
                q_h, k_h, (((1,), (1,)), ((), ())),
                preferred_element_type=F32,
            )
            sc += lax.dot_general(
                qr_h, kr_v, (((1,), (1,)), ((), ())),
                preferred_element_type=F32,
            )
            sc *= SCALE
            m = jnp.max(sc, axis=1, keepdims=True)
            p = jnp.exp(sc - m)
            p = p / jnp.sum(p, axis=1, keepdims=True)
            o_ref[:, j * Dh:(j + 1) * Dh] = jnp.dot(
                p, v_ref[:, j * Dh:(j + 1) * Dh], preferred_element_type=F32
            )

    return pl.pallas_call(
        body,
        grid=(H // HB,),
        in_specs=[
            pl.BlockSpec((s, HB * Dh), lambda g: (0, g)),
            pl.BlockSpec((s, HB * Dh), lambda g: (0, g)),
            pl.BlockSpec((s, HB * Dh), lambda g: (0, g)),
            pl.BlockSpec((s, HB * Dr), lambda g: (0, g)),
            pl.BlockSpec((s, Dr), lambda g: (0, 0)),
        ],
        out_specs=pl.BlockSpec((s, HB * Dh), lambda g: (0, g)),
        out_shape=jax.ShapeDtypeStruct((s, H * Dh), F32),
    )(q, k, v, qr, kr)


def kernel(x, Wdkv, Wuk, Wuv, Wq, Wqr, Wkr, Wo):
    b, s, d = x.shape
    x2 = x.reshape(s, d)

    c = _matmul(x2, Wdkv)
    cg, wukg, wuvg = _gather_z(c, Wuk, Wuv)
    q, qr, kr = _proj_qk(x2, Wq, Wqr, Wkr)
    k = _matmul(cg, wukg)
    v = _matmul(cg, wuvg)
    o = _attention(q, k, v, qr, kr)
    out = _matmul(o, Wo)
    return out.reshape(b, s, d)
